# baseline (speedup 1.0000x reference)
import functools

import jax
import jax.numpy as jnp
from jax.experimental import pallas as pl
from jax.experimental.pallas import tpu as pltpu


def _layer_body(s_ref, x_ref, xt_ref, wa_ref, wb_ref, b_ref, o_ref, *, relu):
    s = s_ref[...].astype(jnp.bfloat16)
    agg = jnp.dot(s, x_ref[...], preferred_element_type=jnp.float32)
    z = jnp.dot(agg.astype(jnp.bfloat16), wa_ref[...],
                preferred_element_type=jnp.float32)
    z = z + jnp.dot(xt_ref[...], wb_ref[...],
                    preferred_element_type=jnp.float32)
    z = z + b_ref[...].astype(jnp.float32)
    if relu:
        z = jnp.maximum(z, 0.0)
    o_ref[...] = z.astype(o_ref.dtype)


def _row_tile(n):
    for r in (400, 200, 80, 40, 16, 8):
        if n % r == 0:
            return r
    return n


def _layer(support, x16, wa, wb, b, *, relu, out_dtype):
    n = support.shape[0]
    d = x16.shape[1]
    h = wa.shape[1]
    r = _row_tile(n)
    grid = (n // r,)
    return pl.pallas_call(
        functools.partial(_layer_body, relu=relu),
        grid=grid,
        in_specs=[
            pl.BlockSpec((r, n), lambda i: (i, 0)),
            pl.BlockSpec((n, d), lambda i: (0, 0)),
            pl.BlockSpec((r, d), lambda i: (i, 0)),
            pl.BlockSpec((d, h), lambda i: (0, 0)),
            pl.BlockSpec((d, h), lambda i: (0, 0)),
            pl.BlockSpec((1, h), lambda i: (0, 0)),
        ],
        out_specs=pl.BlockSpec((r, h), lambda i: (i, 0)),
        out_shape=jax.ShapeDtypeStruct((n, h), out_dtype),
        compiler_params=pltpu.CompilerParams(
            dimension_semantics=("parallel",),
        ),
    )(support, x16, x16, wa, wb, b)


def kernel(support, features, W1, b1, W2, b2):
    n, d_in = features.shape
    h1 = W1.shape[1]
    x16 = features.astype(jnp.bfloat16)
    w1a = W1[:d_in].astype(jnp.bfloat16)
    w1b = W1[d_in:].astype(jnp.bfloat16)
    h = _layer(support, x16, w1a, w1b, b1.reshape(1, -1),
               relu=True, out_dtype=jnp.bfloat16)
    w2a = W2[:h1].astype(jnp.bfloat16)
    w2b = W2[h1:].astype(jnp.bfloat16)
    out = _layer(support, h, w2a, w2b, b2.reshape(1, -1),
                 relu=False, out_dtype=jnp.float32)
    return out


# speedup vs baseline: 1.2497x; 1.2497x over previous
import functools

import numpy as np
import jax
import jax.numpy as jnp
from jax.experimental import pallas as pl
from jax.experimental.pallas import tpu as pltpu

_BLK = 2048


def _gcn_body(i_ref, j_ref, s_blk_ref, f_ref, w1a_ref, w1b_ref, b1_ref,
              w2a_ref, w2b_ref, b2_ref, out_ref,
              acc1_ref, acc2_ref, h1_ref, *, G, P1, n_valid_last):
    s = pl.program_id(0)
    i = i_ref[s]
    j = j_ref[s]
    is_p1 = s < P1
    row_start = is_p1 & (s == i * G)
    is_diag = is_p1 & (j == i)
    R = _BLK

    B = s_blk_ref[...]
    @pl.when(j == G - 1)
    def _mask_cols():
        col = jax.lax.broadcasted_iota(jnp.int32, (R, R), 1)
        s_blk_ref[...] = jnp.where(col < n_valid_last, s_blk_ref[...], 0.0)

    B = s_blk_ref[...].astype(jnp.bfloat16)

    @pl.when(is_p1)
    def _l1():
        fj = f_ref[pl.ds(j * R, R), :]
        contrib = jnp.dot(B, fj, preferred_element_type=jnp.float32)

        @pl.when(row_start)
        def _init():
            acc1_ref[...] = contrib
            acc2_ref[pl.ds(i * R, R), :] = jnp.zeros((R, acc2_ref.shape[1]),
                                                     jnp.float32)

        @pl.when(~row_start)
        def _acc():
            acc1_ref[...] += contrib

    @pl.when(is_diag)
    def _h1():
        fi = f_ref[pl.ds(i * R, R), :]
        z = jnp.dot(acc1_ref[...].astype(jnp.bfloat16), w1a_ref[...],
                    preferred_element_type=jnp.float32)
        z = z + jnp.dot(fi, w1b_ref[...], preferred_element_type=jnp.float32)
        z = z + b1_ref[...].astype(jnp.float32)
        h = jnp.maximum(z, 0.0)
        row = jax.lax.broadcasted_iota(jnp.int32, h.shape, 0)
        h = jnp.where((i < G - 1) | (row < n_valid_last), h, 0.0)
        h1_ref[pl.ds(i * R, R), :] = h.astype(jnp.bfloat16)

    ready = (~is_p1) | (j < i) | is_diag

    @pl.when(ready)
    def _l2():
        hj = h1_ref[pl.ds(j * R, R), :]
        acc2_ref[pl.ds(i * R, R), :] += jnp.dot(
            B, hj, preferred_element_type=jnp.float32)

    last = ((~is_p1) & (j == G - 1)) | (is_diag & (i == G - 1))

    @pl.when(last)
    def _out():
        a2 = acc2_ref[pl.ds(i * R, R), :].astype(jnp.bfloat16)
        hi = h1_ref[pl.ds(i * R, R), :]
        o = jnp.dot(a2, w2a_ref[...], preferred_element_type=jnp.float32)
        o = o + jnp.dot(hi, w2b_ref[...], preferred_element_type=jnp.float32)
        out_ref[...] = o + b2_ref[...].astype(jnp.float32)


def kernel(support, features, W1, b1, W2, b2):
    n, d_in = features.shape
    h1 = W1.shape[1]
    d_out = W2.shape[1]
    G = -(-n // _BLK)
    n_pad = G * _BLK
    P1 = G * G
    n_valid_last = n - (G - 1) * _BLK

    i_tab, j_tab = [], []
    for i in range(G):
        for j in [x for x in range(G) if x != i] + [i]:
            i_tab.append(i)
            j_tab.append(j)
    for i in range(G):
        for j in range(i + 1, G):
            i_tab.append(i)
            j_tab.append(j)
    steps = len(i_tab)
    i_tab = jnp.asarray(np.asarray(i_tab, np.int32))
    j_tab = jnp.asarray(np.asarray(j_tab, np.int32))

    x16 = jnp.zeros((n_pad, d_in), jnp.bfloat16).at[:n].set(
        features.astype(jnp.bfloat16))
    w1a = W1[:d_in].astype(jnp.bfloat16)
    w1b = W1[d_in:].astype(jnp.bfloat16)
    w2a = W2[:h1].astype(jnp.bfloat16)
    w2b = W2[h1:].astype(jnp.bfloat16)

    grid_spec = pltpu.PrefetchScalarGridSpec(
        num_scalar_prefetch=2,
        grid=(steps,),
        in_specs=[
            pl.BlockSpec((_BLK, _BLK), lambda s, it, jt: (it[s], jt[s])),
            pl.BlockSpec((n_pad, d_in), lambda s, it, jt: (0, 0)),
            pl.BlockSpec((d_in, h1), lambda s, it, jt: (0, 0)),
            pl.BlockSpec((d_in, h1), lambda s, it, jt: (0, 0)),
            pl.BlockSpec((1, h1), lambda s, it, jt: (0, 0)),
            pl.BlockSpec((h1, d_out), lambda s, it, jt: (0, 0)),
            pl.BlockSpec((h1, d_out), lambda s, it, jt: (0, 0)),
            pl.BlockSpec((1, d_out), lambda s, it, jt: (0, 0)),
        ],
        out_specs=pl.BlockSpec((_BLK, d_out), lambda s, it, jt: (it[s], 0)),
        scratch_shapes=[
            pltpu.VMEM((_BLK, h1), jnp.float32),
            pltpu.VMEM((n_pad, h1), jnp.float32),
            pltpu.VMEM((n_pad, h1), jnp.bfloat16),
        ],
    )
    return pl.pallas_call(
        functools.partial(_gcn_body, G=G, P1=P1, n_valid_last=n_valid_last),
        grid_spec=grid_spec,
        out_shape=jax.ShapeDtypeStruct((n, d_out), jnp.float32),
        compiler_params=pltpu.CompilerParams(
            dimension_semantics=("arbitrary",),
        ),
    )(i_tab, j_tab, support, x16, w1a, w1b, b1.reshape(1, -1),
      w2a, w2b, b2.reshape(1, -1))


# speedup vs baseline: 1.2947x; 1.0359x over previous
import functools

import numpy as np
import jax
import jax.numpy as jnp
from jax.experimental import pallas as pl
from jax.experimental.pallas import tpu as pltpu

_BLK = 2048


def _gcn_body(i_ref, j_ref, s_blk_ref, fh_ref, w1a_ref, w1b_ref, b1_ref,
              w2a_ref, w2b_ref, b2_ref, out_ref,
              acc1_ref, acc2_ref, *, G, P1, n_valid_last, d_in):
    s = pl.program_id(0)
    i = i_ref[s]
    j = j_ref[s]
    is_p1 = s < P1
    row_start = is_p1 & (s == i * G)
    is_diag = is_p1 & (j == i)
    R = _BLK

    @pl.when(j == G - 1)
    def _mask_cols():
        col = jax.lax.broadcasted_iota(jnp.int32, (R, R), 1)
        s_blk_ref[...] = jnp.where(col < n_valid_last, s_blk_ref[...], 0.0)

    B = s_blk_ref[...].astype(jnp.bfloat16)

    Z = jnp.dot(B, fh_ref[pl.ds(j * R, R), :],
                preferred_element_type=jnp.float32)

    @pl.when(row_start)
    def _assign():
        acc1_ref[...] = Z[:, :d_in]
        acc2_ref[pl.ds(i * R, R), :] = Z[:, d_in:]

    @pl.when(~row_start)
    def _accum():
        acc1_ref[...] += Z[:, :d_in]
        acc2_ref[pl.ds(i * R, R), :] += Z[:, d_in:]

    @pl.when(is_diag)
    def _h1():
        fi = fh_ref[pl.ds(i * R, R), :d_in]
        z = jnp.dot(acc1_ref[...].astype(jnp.bfloat16),
                    w1a_ref[...], preferred_element_type=jnp.float32)
        z = z + jnp.dot(fi, w1b_ref[...], preferred_element_type=jnp.float32)
        z = z + b1_ref[...].astype(jnp.float32)
        h = jnp.maximum(z, 0.0)
        row = jax.lax.broadcasted_iota(jnp.int32, h.shape, 0)
        h = jnp.where((i < G - 1) | (row < n_valid_last), h, 0.0)
        h16 = h.astype(jnp.bfloat16)
        fh_ref[pl.ds(i * R, R), d_in:] = h16
        acc2_ref[pl.ds(i * R, R), :] += jnp.dot(
            B, h16, preferred_element_type=jnp.float32)

    last = ((~is_p1) & (j == G - 1)) | (is_diag & (i == G - 1))

    @pl.when(last)
    def _out():
        a2 = acc2_ref[pl.ds(i * R, R), :].astype(jnp.bfloat16)
        hi = fh_ref[pl.ds(i * R, R), d_in:]
        o = jnp.dot(a2, w2a_ref[...], preferred_element_type=jnp.float32)
        o = o + jnp.dot(hi, w2b_ref[...], preferred_element_type=jnp.float32)
        out_ref[...] = o + b2_ref[...].astype(jnp.float32)


def kernel(support, features, W1, b1, W2, b2):
    n, d_in = features.shape
    h1 = W1.shape[1]
    d_out = W2.shape[1]
    G = -(-n // _BLK)
    n_pad = G * _BLK
    P1 = G * G
    n_valid_last = n - (G - 1) * _BLK

    i_tab, j_tab = [], []
    for i in range(G):
        for j in [x for x in range(G) if x != i] + [i]:
            i_tab.append(i)
            j_tab.append(j)
    for i in range(G):
        for j in range(i + 1, G):
            i_tab.append(i)
            j_tab.append(j)
    steps = len(i_tab)
    i_tab = jnp.asarray(np.asarray(i_tab, np.int32))
    j_tab = jnp.asarray(np.asarray(j_tab, np.int32))

    xh = jnp.zeros((n_pad, d_in + h1), jnp.bfloat16).at[:n, :d_in].set(
        features.astype(jnp.bfloat16))
    w1a = W1[:d_in].astype(jnp.bfloat16)
    w1b = W1[d_in:].astype(jnp.bfloat16)
    w2a = W2[:h1].astype(jnp.bfloat16)
    w2b = W2[h1:].astype(jnp.bfloat16)

    grid_spec = pltpu.PrefetchScalarGridSpec(
        num_scalar_prefetch=2,
        grid=(steps,),
        in_specs=[
            pl.BlockSpec((_BLK, _BLK), lambda s, it, jt: (it[s], jt[s])),
            pl.BlockSpec((n_pad, d_in + h1), lambda s, it, jt: (0, 0)),
            pl.BlockSpec((d_in, h1), lambda s, it, jt: (0, 0)),
            pl.BlockSpec((d_in, h1), lambda s, it, jt: (0, 0)),
            pl.BlockSpec((1, h1), lambda s, it, jt: (0, 0)),
            pl.BlockSpec((h1, d_out), lambda s, it, jt: (0, 0)),
            pl.BlockSpec((h1, d_out), lambda s, it, jt: (0, 0)),
            pl.BlockSpec((1, d_out), lambda s, it, jt: (0, 0)),
        ],
        out_specs=pl.BlockSpec((_BLK, d_out), lambda s, it, jt: (it[s], 0)),
        scratch_shapes=[
            pltpu.VMEM((_BLK, d_in), jnp.float32),
            pltpu.VMEM((n_pad, h1), jnp.float32),
        ],
    )
    return pl.pallas_call(
        functools.partial(_gcn_body, G=G, P1=P1,
                          n_valid_last=n_valid_last, d_in=d_in),
        grid_spec=grid_spec,
        out_shape=jax.ShapeDtypeStruct((n, d_out), jnp.float32),
        compiler_params=pltpu.CompilerParams(
            dimension_semantics=("arbitrary",),
        ),
    )(i_tab, j_tab, support, xh, w1a, w1b, b1.reshape(1, -1),
      w2a, w2b, b2.reshape(1, -1))


# speedup vs baseline: 1.3094x; 1.0113x over previous
import functools

import numpy as np
import jax
import jax.numpy as jnp
from jax.experimental import pallas as pl
from jax.experimental.pallas import tpu as pltpu

_BLK = 2048


def _gcn_body(i_ref, j_ref, st_ref, sb_ref, fh_ref, w1a_ref, w1b_ref,
              b1_ref, w2a_ref, w2b_ref, b2_ref, out_ref,
              acc1_ref, acc2_ref, *, G, P1, n_valid_last, d_in):
    s = pl.program_id(0)
    i = i_ref[s]
    j = j_ref[s]
    is_p1 = s < P1
    row_start = is_p1 & (s == i * G)
    is_diag = is_p1 & (j == i)
    R = _BLK

    @pl.when(j == G - 1)
    def _mask_cols():
        col = jax.lax.broadcasted_iota(jnp.int32, (R // 2, R), 1)
        st_ref[...] = jnp.where(col < n_valid_last, st_ref[...], 0.0)
        sb_ref[...] = jnp.where(col < n_valid_last, sb_ref[...], 0.0)

    Bt = st_ref[...].astype(jnp.bfloat16)
    Bb = sb_ref[...].astype(jnp.bfloat16)

    fhj = fh_ref[pl.ds(j * R, R), :]
    Zt = jnp.dot(Bt, fhj, preferred_element_type=jnp.float32)
    Zb = jnp.dot(Bb, fhj, preferred_element_type=jnp.float32)

    H = R // 2

    @pl.when(row_start)
    def _assign():
        acc1_ref[:H, :] = Zt[:, :d_in]
        acc1_ref[H:, :] = Zb[:, :d_in]
        acc2_ref[pl.ds(i * R, H), :] = Zt[:, d_in:]
        acc2_ref[pl.ds(i * R + H, H), :] = Zb[:, d_in:]

    @pl.when(~row_start)
    def _accum():
        acc1_ref[:H, :] += Zt[:, :d_in]
        acc1_ref[H:, :] += Zb[:, :d_in]
        acc2_ref[pl.ds(i * R, H), :] += Zt[:, d_in:]
        acc2_ref[pl.ds(i * R + H, H), :] += Zb[:, d_in:]

    @pl.when(is_diag)
    def _h1():
        fi = fh_ref[pl.ds(i * R, R), :d_in]
        z = jnp.dot(acc1_ref[...].astype(jnp.bfloat16),
                    w1a_ref[...], preferred_element_type=jnp.float32)
        z = z + jnp.dot(fi, w1b_ref[...], preferred_element_type=jnp.float32)
        z = z + b1_ref[...].astype(jnp.float32)
        h = jnp.maximum(z, 0.0)
        row = jax.lax.broadcasted_iota(jnp.int32, h.shape, 0)
        h = jnp.where((i < G - 1) | (row < n_valid_last), h, 0.0)
        h16 = h.astype(jnp.bfloat16)
        fh_ref[pl.ds(i * R, R), d_in:] = h16
        acc2_ref[pl.ds(i * R, R // 2), :] += jnp.dot(
            Bt, h16, preferred_element_type=jnp.float32)
        acc2_ref[pl.ds(i * R + R // 2, R // 2), :] += jnp.dot(
            Bb, h16, preferred_element_type=jnp.float32)

    last = ((~is_p1) & (j == G - 1)) | (is_diag & (i == G - 1))

    @pl.when(last)
    def _out():
        a2 = acc2_ref[pl.ds(i * R, R), :].astype(jnp.bfloat16)
        hi = fh_ref[pl.ds(i * R, R), d_in:]
        o = jnp.dot(a2, w2a_ref[...], preferred_element_type=jnp.float32)
        o = o + jnp.dot(hi, w2b_ref[...], preferred_element_type=jnp.float32)
        out_ref[...] = o + b2_ref[...].astype(jnp.float32)


def kernel(support, features, W1, b1, W2, b2):
    n, d_in = features.shape
    h1 = W1.shape[1]
    d_out = W2.shape[1]
    G = -(-n // _BLK)
    n_pad = G * _BLK
    P1 = G * G
    n_valid_last = n - (G - 1) * _BLK

    i_tab, j_tab = [], []
    for i in range(G):
        for j in [x for x in range(G) if x != i] + [i]:
            i_tab.append(i)
            j_tab.append(j)
    for i in range(G):
        for j in range(i + 1, G):
            i_tab.append(i)
            j_tab.append(j)
    steps = len(i_tab)
    i_tab = jnp.asarray(np.asarray(i_tab, np.int32))
    j_tab = jnp.asarray(np.asarray(j_tab, np.int32))

    xh = jnp.zeros((n_pad, d_in + h1), jnp.bfloat16).at[:n, :d_in].set(
        features.astype(jnp.bfloat16))
    w1a = W1[:d_in].astype(jnp.bfloat16)
    w1b = W1[d_in:].astype(jnp.bfloat16)
    w2a = W2[:h1].astype(jnp.bfloat16)
    w2b = W2[h1:].astype(jnp.bfloat16)

    grid_spec = pltpu.PrefetchScalarGridSpec(
        num_scalar_prefetch=2,
        grid=(steps,),
        in_specs=[
            pl.BlockSpec((_BLK // 2, _BLK),
                         lambda s, it, jt: (2 * it[s], jt[s])),
            pl.BlockSpec((_BLK // 2, _BLK),
                         lambda s, it, jt: (2 * it[s] + 1, jt[s])),
            pl.BlockSpec((n_pad, d_in + h1), lambda s, it, jt: (0, 0)),
            pl.BlockSpec((d_in, h1), lambda s, it, jt: (0, 0)),
            pl.BlockSpec((d_in, h1), lambda s, it, jt: (0, 0)),
            pl.BlockSpec((1, h1), lambda s, it, jt: (0, 0)),
            pl.BlockSpec((h1, d_out), lambda s, it, jt: (0, 0)),
            pl.BlockSpec((h1, d_out), lambda s, it, jt: (0, 0)),
            pl.BlockSpec((1, d_out), lambda s, it, jt: (0, 0)),
        ],
        out_specs=pl.BlockSpec((_BLK, d_out), lambda s, it, jt: (it[s], 0)),
        scratch_shapes=[
            pltpu.VMEM((_BLK, d_in), jnp.float32),
            pltpu.VMEM((n_pad, h1), jnp.float32),
        ],
    )
    return pl.pallas_call(
        functools.partial(_gcn_body, G=G, P1=P1,
                          n_valid_last=n_valid_last, d_in=d_in),
        grid_spec=grid_spec,
        out_shape=jax.ShapeDtypeStruct((n, d_out), jnp.float32),
        compiler_params=pltpu.CompilerParams(
            dimension_semantics=("arbitrary",),
        ),
    )(i_tab, j_tab, support, support, xh, w1a, w1b, b1.reshape(1, -1),
      w2a, w2b, b2.reshape(1, -1))


# speedup vs baseline: 1.3294x; 1.0153x over previous
import functools

import numpy as np
import jax
import jax.numpy as jnp
from jax.experimental import pallas as pl
from jax.experimental.pallas import tpu as pltpu

_BLK = 2048


def _gcn_body(i_ref, j_ref, st_ref, sb_ref, fh_ref, w1a_ref, w1b_ref,
              b1_ref, w2a_ref, w2b_ref, b2_ref, out_ref,
              acc1_ref, acc2_ref, *, G, P1, n_valid_last, d_in):
    s = pl.program_id(0)
    i = i_ref[s]
    j = j_ref[s]
    is_p1 = s < P1
    row_start = is_p1 & (s == i * G)
    is_diag = is_p1 & (j == i)
    R = _BLK

    Bt = st_ref[...].astype(jnp.bfloat16)
    Bb = sb_ref[...].astype(jnp.bfloat16)

    fhj = fh_ref[pl.ds(j * R, R), :]
    Zt = jnp.dot(Bt, fhj, preferred_element_type=jnp.float32)
    Zb = jnp.dot(Bb, fhj, preferred_element_type=jnp.float32)

    H = R // 2

    @pl.when(row_start)
    def _assign():
        acc1_ref[:H, :] = Zt[:, :d_in]
        acc1_ref[H:, :] = Zb[:, :d_in]
        acc2_ref[pl.ds(i * R, H), :] = Zt[:, d_in:]
        acc2_ref[pl.ds(i * R + H, H), :] = Zb[:, d_in:]

    @pl.when(~row_start)
    def _accum():
        acc1_ref[:H, :] += Zt[:, :d_in]
        acc1_ref[H:, :] += Zb[:, :d_in]
        acc2_ref[pl.ds(i * R, H), :] += Zt[:, d_in:]
        acc2_ref[pl.ds(i * R + H, H), :] += Zb[:, d_in:]

    @pl.when(is_diag)
    def _h1():
        fi = fh_ref[pl.ds(i * R, R), :d_in]
        z = jnp.dot(acc1_ref[...].astype(jnp.bfloat16),
                    w1a_ref[...], preferred_element_type=jnp.float32)
        z = z + jnp.dot(fi, w1b_ref[...], preferred_element_type=jnp.float32)
        z = z + b1_ref[...].astype(jnp.float32)
        h = jnp.maximum(z, 0.0)
        row = jax.lax.broadcasted_iota(jnp.int32, h.shape, 0)
        h = jnp.where((i < G - 1) | (row < n_valid_last), h, 0.0)
        h16 = h.astype(jnp.bfloat16)
        fh_ref[pl.ds(i * R, R), d_in:] = h16
        acc2_ref[pl.ds(i * R, R // 2), :] += jnp.dot(
            Bt, h16, preferred_element_type=jnp.float32)
        acc2_ref[pl.ds(i * R + R // 2, R // 2), :] += jnp.dot(
            Bb, h16, preferred_element_type=jnp.float32)

    last = ((~is_p1) & (j == G - 1)) | (is_diag & (i == G - 1))

    @pl.when(last)
    def _out():
        a2 = acc2_ref[pl.ds(i * R, R), :].astype(jnp.bfloat16)
        hi = fh_ref[pl.ds(i * R, R), d_in:]
        o = jnp.dot(a2, w2a_ref[...], preferred_element_type=jnp.float32)
        o = o + jnp.dot(hi, w2b_ref[...], preferred_element_type=jnp.float32)
        out_ref[...] = o + b2_ref[...].astype(jnp.float32)


def kernel(support, features, W1, b1, W2, b2):
    n, d_in = features.shape
    h1 = W1.shape[1]
    d_out = W2.shape[1]
    G = -(-n // _BLK)
    n_pad = G * _BLK
    P1 = G * G
    n_valid_last = n - (G - 1) * _BLK

    i_tab, j_tab = [], []
    for i in range(G):
        for j in [x for x in range(G) if x != i] + [i]:
            i_tab.append(i)
            j_tab.append(j)
    for i in range(G):
        for j in range(i + 1, G):
            i_tab.append(i)
            j_tab.append(j)
    steps = len(i_tab)
    i_tab = jnp.asarray(np.asarray(i_tab, np.int32))
    j_tab = jnp.asarray(np.asarray(j_tab, np.int32))

    xh = jnp.zeros((n_pad, d_in + h1), jnp.bfloat16).at[:n, :d_in].set(
        features.astype(jnp.bfloat16))
    w1a = W1[:d_in].astype(jnp.bfloat16)
    w1b = W1[d_in:].astype(jnp.bfloat16)
    w2a = W2[:h1].astype(jnp.bfloat16)
    w2b = W2[h1:].astype(jnp.bfloat16)

    grid_spec = pltpu.PrefetchScalarGridSpec(
        num_scalar_prefetch=2,
        grid=(steps,),
        in_specs=[
            pl.BlockSpec((_BLK // 2, _BLK),
                         lambda s, it, jt: (2 * it[s], jt[s])),
            pl.BlockSpec((_BLK // 2, _BLK),
                         lambda s, it, jt: (2 * it[s] + 1, jt[s])),
            pl.BlockSpec((n_pad, d_in + h1), lambda s, it, jt: (0, 0)),
            pl.BlockSpec((d_in, h1), lambda s, it, jt: (0, 0)),
            pl.BlockSpec((d_in, h1), lambda s, it, jt: (0, 0)),
            pl.BlockSpec((1, h1), lambda s, it, jt: (0, 0)),
            pl.BlockSpec((h1, d_out), lambda s, it, jt: (0, 0)),
            pl.BlockSpec((h1, d_out), lambda s, it, jt: (0, 0)),
            pl.BlockSpec((1, d_out), lambda s, it, jt: (0, 0)),
        ],
        out_specs=pl.BlockSpec((_BLK, d_out), lambda s, it, jt: (it[s], 0)),
        scratch_shapes=[
            pltpu.VMEM((_BLK, d_in), jnp.float32),
            pltpu.VMEM((n_pad, h1), jnp.float32),
        ],
    )
    return pl.pallas_call(
        functools.partial(_gcn_body, G=G, P1=P1,
                          n_valid_last=n_valid_last, d_in=d_in),
        grid_spec=grid_spec,
        out_shape=jax.ShapeDtypeStruct((n, d_out), jnp.float32),
        compiler_params=pltpu.CompilerParams(
            dimension_semantics=("arbitrary",),
        ),
    )(i_tab, j_tab, support, support, xh, w1a, w1b, b1.reshape(1, -1),
      w2a, w2b, b2.reshape(1, -1))
